# baseline (speedup 1.0000x reference)
import functools

import jax
import jax.numpy as jnp
from jax import lax
from jax.experimental import pallas as pl
from jax.experimental.pallas import tpu as pltpu
from jax.experimental.pallas import tpu_sc as plsc

B = 16384
K = 32
NC = 2
NS = 16
NW = NC * NS
BPW = B // NW


def _sc_gather(uidx, iidx, W, H, ubf, ibf):
    mesh = plsc.VectorSubcoreMesh(core_axis_name="c", subcore_axis_name="s")

    @functools.partial(
        pl.kernel,
        out_type=(
            jax.ShapeDtypeStruct((B, K), jnp.float32),
            jax.ShapeDtypeStruct((B, K), jnp.float32),
            jax.ShapeDtypeStruct((B,), jnp.float32),
            jax.ShapeDtypeStruct((B,), jnp.float32),
        ),
        mesh=mesh,
        scratch_types=[
            pltpu.VMEM((BPW,), jnp.int32),
            pltpu.VMEM((BPW,), jnp.int32),
            pltpu.VMEM((BPW, K), jnp.float32),
            pltpu.VMEM((BPW, K), jnp.float32),
            pltpu.VMEM((BPW,), jnp.float32),
            pltpu.VMEM((BPW,), jnp.float32),
            pltpu.SemaphoreType.DMA,
        ],
        compiler_params=pltpu.CompilerParams(use_tc_tiling_on_sc=False),
    )
    def gather_kernel(uidx_hbm, iidx_hbm, w_hbm, h_hbm, ub_hbm, ib_hbm,
                      uo_hbm, vo_hbm, ubo_hbm, ibo_hbm,
                      uidx_v, iidx_v, urows_v, vrows_v, ubg_v, ibg_v, sem):
        wid = lax.axis_index("s") * NC + lax.axis_index("c")
        base = wid * BPW
        pltpu.sync_copy(uidx_hbm.at[pl.ds(base, BPW)], uidx_v)
        pltpu.sync_copy(iidx_hbm.at[pl.ds(base, BPW)], iidx_v)
        cps = (
            pltpu.async_copy(w_hbm.at[uidx_v], urows_v, sem),
            pltpu.async_copy(h_hbm.at[iidx_v], vrows_v, sem),
            pltpu.async_copy(ub_hbm.at[uidx_v], ubg_v, sem),
            pltpu.async_copy(ib_hbm.at[iidx_v], ibg_v, sem),
        )
        for cp in cps:
            cp.wait()
        pltpu.sync_copy(urows_v, uo_hbm.at[pl.ds(base, BPW)])
        pltpu.sync_copy(vrows_v, vo_hbm.at[pl.ds(base, BPW)])
        pltpu.sync_copy(ubg_v, ubo_hbm.at[pl.ds(base, BPW)])
        pltpu.sync_copy(ibg_v, ibo_hbm.at[pl.ds(base, BPW)])

    return gather_kernel(uidx, iidx, W, H, ubf, ibf)


def _mlp_body(u_ref, v_ref, ub_ref, ib_ref, w1a_ref, w1b_ref, b1_ref, w2_ref,
              o_ref):
    h = (
        jnp.dot(u_ref[...], w1a_ref[...], preferred_element_type=jnp.float32)
        + jnp.dot(v_ref[...], w1b_ref[...], preferred_element_type=jnp.float32)
        + b1_ref[...]
    )
    h = jnp.maximum(h, 0.0)
    acc = jnp.sum(h * w2_ref[...], axis=1, keepdims=True)
    o_ref[...] = acc + ub_ref[...] + ib_ref[...]


def _tc_mlp(U, V, ubg, ibg, w1a, w1b, b1r, w2r):
    blk = 2048
    grid = (B // blk,)
    return pl.pallas_call(
        _mlp_body,
        grid=grid,
        in_specs=[
            pl.BlockSpec((blk, K), lambda i: (i, 0)),
            pl.BlockSpec((blk, K), lambda i: (i, 0)),
            pl.BlockSpec((blk, 1), lambda i: (i, 0)),
            pl.BlockSpec((blk, 1), lambda i: (i, 0)),
            pl.BlockSpec((K, K), lambda i: (0, 0)),
            pl.BlockSpec((K, K), lambda i: (0, 0)),
            pl.BlockSpec((1, K), lambda i: (0, 0)),
            pl.BlockSpec((1, K), lambda i: (0, 0)),
        ],
        out_specs=pl.BlockSpec((blk, 1), lambda i: (i, 0)),
        out_shape=jax.ShapeDtypeStruct((B, 1), jnp.float32),
    )(U, V, ubg, ibg, w1a, w1b, b1r, w2r)


def kernel(x, W, H, W_pre, H_pre, W_eps, H_eps, W1, b1, W2, ub, ib):
    uidx = x[:, 0]
    iidx = x[:, 1]
    U, V, ubg, ibg = _sc_gather(uidx, iidx, W, H,
                                ub.reshape(-1), ib.reshape(-1))
    w1a = W1[:, :K].T
    w1b = W1[:, K:].T
    return _tc_mlp(U, V, ubg.reshape(B, 1), ibg.reshape(B, 1),
                   w1a, w1b, b1.reshape(1, K), W2)
